# Initial kernel scaffold; baseline (speedup 1.0000x reference)
import jax
import jax.numpy as jnp
from jax.experimental import pallas as pl


def kernel(texts, embed_table, posit_table, W, b):
    raise NotImplementedError("write your pallas kernel here")


# speedup vs baseline: 89.4529x; 89.4529x over previous
import functools

import jax
import jax.numpy as jnp
from jax import lax
from jax.experimental import pallas as pl
from jax.experimental.pallas import tpu as pltpu
from jax.experimental.pallas import tpu_sc as plsc

VOCAB_PAD = 30720
EMB_DIM = 128
SEQ = 200
BATCH = 16384
NCLS = 3

NUM_WORKERS = 32
ROWS_PER_WORKER = BATCH // NUM_WORKERS
CHUNK = 64
NCHUNK = ROWS_PER_WORKER // CHUNK
TABLE_WORDS = VOCAB_PAD * NCLS
TXT_WORDS = SEQ * CHUNK
OUT_STRIDE = 256


def _table_body(e_ref, p_ref, w_ref, b_ref, o_ref):
    w = w_ref[...]
    pmw = jnp.dot(p_ref[...], w, preferred_element_type=jnp.float32)
    cst = jnp.mean(pmw, axis=0, keepdims=True) + b_ref[...]
    ew = jnp.dot(e_ref[...], w, preferred_element_type=jnp.float32)
    o_ref[...] = (ew + cst) * (1.0 / SEQ)


_build_table = pl.pallas_call(
    _table_body,
    grid=(15,),
    in_specs=[
        pl.BlockSpec((2048, EMB_DIM), lambda i: (i, 0)),
        pl.BlockSpec((SEQ, EMB_DIM), lambda i: (0, 0)),
        pl.BlockSpec((EMB_DIM, NCLS), lambda i: (0, 0)),
        pl.BlockSpec((1, NCLS), lambda i: (0, 0)),
    ],
    out_specs=pl.BlockSpec((2048, NCLS), lambda i: (i, 0)),
    out_shape=jax.ShapeDtypeStruct((VOCAB_PAD, NCLS), jnp.float32),
)


@functools.partial(
    pl.kernel,
    out_type=jax.ShapeDtypeStruct((BATCH // CHUNK * OUT_STRIDE,), jnp.float32),
    mesh=plsc.VectorSubcoreMesh(core_axis_name="c", subcore_axis_name="s"),
    compiler_params=pltpu.CompilerParams(needs_layout_passes=False),
    scratch_types=[
        pltpu.VMEM((TABLE_WORDS,), jnp.float32),
        pltpu.VMEM((2 * TXT_WORDS,), jnp.int32),
        pltpu.VMEM((NCHUNK * OUT_STRIDE,), jnp.float32),
        pltpu.SemaphoreType.DMA,
        pltpu.SemaphoreType.DMA,
        pltpu.SemaphoreType.DMA,
        pltpu.SemaphoreType.DMA,
    ],
)
def _sc_pool(table_hbm, texts_hbm, out_hbm, table_v, txt_v, out_v,
             tsem, in_sem0, in_sem1, osem):
    cid = lax.axis_index("c")
    sid = lax.axis_index("s")
    wid = sid * 2 + cid

    tcopy = pltpu.make_async_copy(table_hbm, table_v, tsem)
    tcopy.start()

    in_sems = [in_sem0, in_sem1]
    handles = [None] * NCHUNK
    handles[0] = pltpu.make_async_copy(
        texts_hbm.at[pl.ds(wid * (NCHUNK * TXT_WORDS), TXT_WORDS)],
        txt_v.at[pl.ds(0, TXT_WORDS)], in_sems[0])
    handles[0].start()
    tcopy.wait()

    iota = lax.iota(jnp.int32, 16)
    zero = jnp.zeros((16,), jnp.float32)

    for c in range(NCHUNK):
        if c + 1 < NCHUNK:
            handles[c + 1] = pltpu.make_async_copy(
                texts_hbm.at[pl.ds((wid * NCHUNK + c + 1) * TXT_WORDS, TXT_WORDS)],
                txt_v.at[pl.ds(((c + 1) % 2) * TXT_WORDS, TXT_WORDS)],
                in_sems[(c + 1) % 2])
            handles[c + 1].start()
        handles[c].wait()
        buf_base = (c % 2) * TXT_WORDS

        def body(t, carry):
            new = []
            for j in range(CHUNK // 16):
                tok = txt_v[pl.ds(buf_base + t * CHUNK + 16 * j, 16)]
                idx = tok * 3
                g0 = plsc.load_gather(table_v, [idx])
                g1 = plsc.load_gather(table_v, [idx + 1])
                g2 = plsc.load_gather(table_v, [idx + 2])
                a0, a1, a2 = carry[3 * j], carry[3 * j + 1], carry[3 * j + 2]
                new.extend([a0 + g0, a1 + g1, a2 + g2])
            return tuple(new)

        accs = lax.fori_loop(0, SEQ, body, (zero,) * (3 * CHUNK // 16))

        for j in range(CHUNK // 16):
            base = iota * 3 + (c * OUT_STRIDE + j * 48)
            plsc.store_scatter(out_v, [base], accs[3 * j])
            plsc.store_scatter(out_v, [base + 1], accs[3 * j + 1])
            plsc.store_scatter(out_v, [base + 2], accs[3 * j + 2])
        pltpu.make_async_copy(
            out_v.at[pl.ds(c * OUT_STRIDE, OUT_STRIDE)],
            out_hbm.at[pl.ds((wid * NCHUNK + c) * OUT_STRIDE, OUT_STRIDE)],
            osem).start()

    for c in range(NCHUNK):
        pltpu.make_async_copy(
            out_v.at[pl.ds(c * OUT_STRIDE, OUT_STRIDE)],
            out_hbm.at[pl.ds((wid * NCHUNK + c) * OUT_STRIDE, OUT_STRIDE)],
            osem).wait()


def kernel(texts, embed_table, posit_table, W, b):
    e_pad = jnp.pad(embed_table, ((0, VOCAB_PAD - embed_table.shape[0]), (0, 0)))
    table = _build_table(e_pad, posit_table, W, b.reshape(1, NCLS))
    texts_re = (texts.reshape(NUM_WORKERS, NCHUNK, CHUNK, SEQ)
                .transpose(0, 1, 3, 2)
                .reshape(-1))
    out_flat = _sc_pool(table.reshape(-1), texts_re)
    out = out_flat.reshape(BATCH // CHUNK, OUT_STRIDE)[:, :CHUNK * NCLS]
    return out.reshape(BATCH, NCLS)


# speedup vs baseline: 101.7059x; 1.1370x over previous
import functools

import jax
import jax.numpy as jnp
from jax import lax
from jax.experimental import pallas as pl
from jax.experimental.pallas import tpu as pltpu
from jax.experimental.pallas import tpu_sc as plsc

VOCAB = 30522
VOCAB_PAD = 30720
EMB_DIM = 128
SEQ = 200
BATCH = 16384
NCLS = 3

NUM_WORKERS = 32
ROWS_PER_WORKER = BATCH // NUM_WORKERS
CHUNK = 64
NCHUNK = ROWS_PER_WORKER // CHUNK
TABLE_WORDS = VOCAB_PAD * NCLS
TXT_WORDS = SEQ * CHUNK
TBLK = 2048


def _table_body(e_ref, p_ref, w_ref, b_ref, o_ref):
    w = w_ref[...]
    pmw = jnp.dot(p_ref[...], w, preferred_element_type=jnp.float32)
    cst = jnp.mean(pmw, axis=0, keepdims=True) + b_ref[...]
    ew = jnp.dot(e_ref[...], w, preferred_element_type=jnp.float32)
    val = (ew + cst) * (1.0 / SEQ)
    row = pl.program_id(0) * TBLK + lax.broadcasted_iota(jnp.int32, (TBLK, NCLS), 0)
    o_ref[...] = jnp.where(row < VOCAB, val, 0.0)


_build_table = pl.pallas_call(
    _table_body,
    grid=(VOCAB_PAD // TBLK,),
    in_specs=[
        pl.BlockSpec((TBLK, EMB_DIM), lambda i: (i, 0)),
        pl.BlockSpec((SEQ, EMB_DIM), lambda i: (0, 0)),
        pl.BlockSpec((EMB_DIM, NCLS), lambda i: (0, 0)),
        pl.BlockSpec((1, NCLS), lambda i: (0, 0)),
    ],
    out_specs=pl.BlockSpec((TBLK, NCLS), lambda i: (i, 0)),
    out_shape=jax.ShapeDtypeStruct((VOCAB_PAD, NCLS), jnp.float32),
)


@functools.partial(
    pl.kernel,
    out_type=jax.ShapeDtypeStruct((BATCH * NCLS,), jnp.float32),
    mesh=plsc.VectorSubcoreMesh(core_axis_name="c", subcore_axis_name="s"),
    compiler_params=pltpu.CompilerParams(needs_layout_passes=False),
    scratch_types=[
        pltpu.VMEM((TABLE_WORDS,), jnp.float32),
        pltpu.VMEM((TXT_WORDS + 16,), jnp.int32),
        pltpu.VMEM((TXT_WORDS + 16,), jnp.int32),
        pltpu.VMEM((ROWS_PER_WORKER * NCLS,), jnp.float32),
        pltpu.SemaphoreType.DMA,
        pltpu.SemaphoreType.DMA,
        pltpu.SemaphoreType.DMA,
        pltpu.SemaphoreType.DMA,
    ],
)
def _sc_pool(table_hbm, texts_hbm, out_hbm, table_v, txt_a, txt_b, out_v,
             tsem, sem_a, sem_b, osem):
    cid = lax.axis_index("c")
    sid = lax.axis_index("s")
    wid = sid * 2 + cid

    tcopy = pltpu.make_async_copy(table_hbm, table_v, tsem)
    tcopy.start()

    bufs = [txt_a, txt_b]
    sems = [sem_a, sem_b]
    handles = [None] * NCHUNK
    handles[0] = pltpu.make_async_copy(
        texts_hbm.at[pl.ds(wid * (NCHUNK * TXT_WORDS), TXT_WORDS)],
        txt_a.at[pl.ds(0, TXT_WORDS)], sem_a)
    handles[0].start()
    tcopy.wait()

    lane = lax.iota(jnp.int32, 16)
    lane0 = lane == 0
    rem_mask = lane < 8
    zero = jnp.zeros((16,), jnp.float32)
    pad_tok = jnp.full((16,), VOCAB, jnp.int32)

    for c in range(NCHUNK):
        if c + 1 < NCHUNK:
            handles[c + 1] = pltpu.make_async_copy(
                texts_hbm.at[pl.ds((wid * NCHUNK + c + 1) * TXT_WORDS, TXT_WORDS)],
                bufs[(c + 1) % 2].at[pl.ds(0, TXT_WORDS)], sems[(c + 1) % 2])
            handles[c + 1].start()
        handles[c].wait()
        txt = bufs[c % 2]

        def body(r, carry):
            a0 = a1 = a2 = zero
            for j in range(13):
                tok = txt[pl.ds(r * SEQ + 16 * j, 16)]
                if j == 12:
                    tok = jnp.where(rem_mask, tok, pad_tok)
                idx = tok * 3
                a0 = a0 + plsc.load_gather(table_v, [idx])
                a1 = a1 + plsc.load_gather(table_v, [idx + 1])
                a2 = a2 + plsc.load_gather(table_v, [idx + 2])
            base = (c * CHUNK + r) * NCLS
            for k, a in enumerate((a0, a1, a2)):
                s = jnp.full((16,), jnp.sum(a), jnp.float32)
                plsc.store_scatter(out_v, [jnp.full((16,), base + k, jnp.int32)],
                                   s, mask=lane0)
            return carry

        lax.fori_loop(0, CHUNK, body, 0)

    pltpu.make_async_copy(
        out_v,
        out_hbm.at[pl.ds(wid * (ROWS_PER_WORKER * NCLS), ROWS_PER_WORKER * NCLS)],
        osem).start()
    pltpu.make_async_copy(
        out_v,
        out_hbm.at[pl.ds(wid * (ROWS_PER_WORKER * NCLS), ROWS_PER_WORKER * NCLS)],
        osem).wait()


def kernel(texts, embed_table, posit_table, W, b):
    table = _build_table(embed_table, posit_table, W, b.reshape(1, NCLS))
    out_flat = _sc_pool(table.reshape(-1), texts.reshape(-1))
    return out_flat.reshape(BATCH, NCLS)


# speedup vs baseline: 268.2901x; 2.6379x over previous
import functools

import jax
import jax.numpy as jnp
from jax import lax
from jax.experimental import pallas as pl
from jax.experimental.pallas import tpu as pltpu
from jax.experimental.pallas import tpu_sc as plsc

VOCAB = 30522
VOCAB_PAD = 30720
EMB_DIM = 128
SEQ = 200
BATCH = 16384
NCLS = 3

NUM_WORKERS = 32
ROWS_PER_WORKER = BATCH // NUM_WORKERS
CHUNK = 128
NCHUNK = ROWS_PER_WORKER // CHUNK
HALVES = ((0, 96), (96, 104))
TBLK = 7680


def _table_body(e_ref, p_ref, wt_ref, b_ref, o_ref):
    wt = wt_ref[...]
    pmw = lax.dot_general(wt, p_ref[...], (((1,), (1,)), ((), ())),
                          preferred_element_type=jnp.float32)
    cst = jnp.mean(pmw, axis=1, keepdims=True) + b_ref[...]
    ew = lax.dot_general(wt.astype(jnp.bfloat16),
                         e_ref[...].astype(jnp.bfloat16),
                         (((1,), (1,)), ((), ())),
                         preferred_element_type=jnp.float32)
    val = (ew + cst) * (1.0 / SEQ)
    col = pl.program_id(0) * TBLK + lax.broadcasted_iota(jnp.int32, (NCLS, TBLK), 1)
    val = jnp.where(col < VOCAB, val, 0.0)
    u0 = lax.bitcast_convert_type(
        val[0:1].astype(jnp.bfloat16), jnp.uint16).astype(jnp.uint32)
    u1 = lax.bitcast_convert_type(
        val[1:2].astype(jnp.bfloat16), jnp.uint16).astype(jnp.uint32)
    packed = lax.bitcast_convert_type(u0 | (u1 << 16), jnp.float32)
    o_ref[...] = jnp.concatenate([packed, val[2:3]], axis=0)


_build_table = pl.pallas_call(
    _table_body,
    grid=(VOCAB_PAD // TBLK,),
    in_specs=[
        pl.BlockSpec((TBLK, EMB_DIM), lambda i: (i, 0)),
        pl.BlockSpec((SEQ, EMB_DIM), lambda i: (0, 0)),
        pl.BlockSpec((NCLS, EMB_DIM), lambda i: (0, 0)),
        pl.BlockSpec((NCLS, 1), lambda i: (0, 0)),
    ],
    out_specs=pl.BlockSpec((2, TBLK), lambda i: (0, i)),
    out_shape=jax.ShapeDtypeStruct((2, VOCAB_PAD), jnp.float32),
)


@functools.partial(
    pl.kernel,
    out_type=jax.ShapeDtypeStruct((NCLS * BATCH,), jnp.float32),
    mesh=plsc.VectorSubcoreMesh(core_axis_name="c", subcore_axis_name="s"),
    compiler_params=pltpu.CompilerParams(needs_layout_passes=False),
    scratch_types=[
        pltpu.VMEM((2 * VOCAB_PAD,), jnp.float32),
        pltpu.VMEM((2, 104, CHUNK), jnp.int32),
        pltpu.VMEM((NCLS * ROWS_PER_WORKER,), jnp.float32),
        pltpu.SemaphoreType.DMA,
        pltpu.SemaphoreType.DMA,
        pltpu.SemaphoreType.DMA,
        pltpu.SemaphoreType.DMA,
    ],
)
def _sc_pool(table_hbm, texts_hbm, out_hbm, table_v, txt_v, out_v,
             tsem, sem_a, sem_b, osem):
    cid = lax.axis_index("c")
    sid = lax.axis_index("s")
    wid = sid * 2 + cid
    col0 = wid * ROWS_PER_WORKER

    tcopy = pltpu.make_async_copy(table_hbm, table_v, tsem)
    tcopy.start()

    sems = [sem_a, sem_b]
    steps = [(c, h) for c in range(NCHUNK) for h in HALVES]

    def make_copy(s, buf):
        c, (p0, np_) = steps[s]
        return pltpu.make_async_copy(
            texts_hbm.at[pl.ds(p0, np_), pl.ds(col0 + c * CHUNK, CHUNK)],
            txt_v.at[buf, pl.ds(0, np_), :], sems[buf])

    handles = [None] * len(steps)
    handles[0] = make_copy(0, 0)
    handles[0].start()
    tcopy.wait()

    zero = jnp.zeros((16,), jnp.float32)
    accs = None

    for s, (c, (p0, np_)) in enumerate(steps):
        if s + 1 < len(steps):
            handles[s + 1] = make_copy(s + 1, (s + 1) % 2)
            handles[s + 1].start()
        handles[s].wait()
        buf = s % 2
        if p0 == 0:
            accs = (zero,) * (3 * CHUNK // 16)

        hi_mask = jnp.full((16,), -65536, jnp.int32)

        def body(t, carry):
            new = []
            for g in range(CHUNK // 16):
                tok = txt_v[buf, t, pl.ds(16 * g, 16)]
                g01 = plsc.bitcast(plsc.load_gather(table_v, [tok]), jnp.int32)
                g2 = plsc.load_gather(table_v, [tok + VOCAB_PAD])
                c0 = plsc.bitcast(g01 << 16, jnp.float32)
                c1 = plsc.bitcast(g01 & hi_mask, jnp.float32)
                new.extend([carry[3 * g] + c0, carry[3 * g + 1] + c1,
                            carry[3 * g + 2] + g2])
            return tuple(new)

        accs = lax.fori_loop(0, np_, body, accs, unroll=2)

        if p0 != 0:
            for g in range(CHUNK // 16):
                for k in range(NCLS):
                    out_v[pl.ds(k * ROWS_PER_WORKER + c * CHUNK + 16 * g, 16)] = (
                        accs[3 * g + k])

    for k in range(NCLS):
        pltpu.make_async_copy(
            out_v.at[pl.ds(k * ROWS_PER_WORKER, ROWS_PER_WORKER)],
            out_hbm.at[pl.ds(k * BATCH + wid * ROWS_PER_WORKER,
                             ROWS_PER_WORKER)],
            osem).start()
    for k in range(NCLS):
        pltpu.make_async_copy(
            out_v.at[pl.ds(k * ROWS_PER_WORKER, ROWS_PER_WORKER)],
            out_hbm.at[pl.ds(k * BATCH + wid * ROWS_PER_WORKER,
                             ROWS_PER_WORKER)],
            osem).wait()


def kernel(texts, embed_table, posit_table, W, b):
    table3 = _build_table(embed_table, posit_table, W.T, b.reshape(NCLS, 1))
    out_flat = _sc_pool(table3.reshape(-1), texts.T)
    return out_flat.reshape(NCLS, BATCH).T
